# baseline (speedup 1.0000x reference)
import functools

import jax
import jax.numpy as jnp
from jax import lax
from jax.experimental import pallas as pl
from jax.experimental.pallas import tpu as pltpu
from jax.experimental.pallas import tpu_sc as plsc

N = 10000
NPAD = 10240
E = 160000
NW = 32
CH = 128
NCHUNK = 40
EPAD = NW * NCHUNK * CH
RPT = NPAD // 16
RB = 1024
GRID = NPAD // RB


@functools.cache
def _mesh():
    return plsc.VectorSubcoreMesh(
        core_axis_name="c", subcore_axis_name="s", num_cores=2, num_subcores=16)


def _sc_degree_body(col_hbm, ones_hbm, zeros_hbm, out_hbm, colv, onev, degs):
    cid = lax.axis_index("c")
    sid = lax.axis_index("s")
    wid = cid * 16 + sid
    sl = pl.ds(sid * RPT, RPT)
    pltpu.sync_copy(zeros_hbm.at[sl], degs.at[sl])
    pltpu.sync_copy(col_hbm.at[pl.ds(wid * NCHUNK, NCHUNK)], colv)
    pltpu.sync_copy(ones_hbm, onev)
    plsc.subcore_barrier()

    def body(j, carry):
        pltpu.sync_copy(onev, degs.at[colv.at[j]], add=True)
        return carry

    lax.fori_loop(0, NCHUNK, body, 0)
    plsc.subcore_barrier()
    pltpu.sync_copy(degs.at[sl], out_hbm.at[cid, sl])


@functools.cache
def _sc_degree_kernel():
    return pl.kernel(
        _sc_degree_body,
        out_type=jax.ShapeDtypeStruct((2, NPAD, 16), jnp.float32),
        mesh=_mesh(),
        scratch_types=[
            pltpu.VMEM((NCHUNK, CH), jnp.int32),
            pltpu.VMEM((CH, 16), jnp.float32),
            pltpu.VMEM_SHARED((NPAD, 16), jnp.float32),
        ],
    )


def _sc_degree(col, ones16, zeros16):
    return _sc_degree_kernel()(col, ones16, zeros16)


C0 = 16
C1 = 64
CMAX = max(C0, C1)


def _sc_aggregate_body(q_hbm, row_hbm, col_hbm, zeros_hbm, out_hbm,
                       rowv, colv, gbuf, aggs, sem):
    cid = lax.axis_index("c")
    sid = lax.axis_index("s")
    sl = pl.ds(sid * RPT, RPT)
    pltpu.sync_copy(zeros_hbm.at[sl], aggs.at[sl])

    start = jnp.where(cid == 0, sid * C0, 16 * C0 + sid * C1)
    start = pl.multiple_of(start, 8)
    pltpu.sync_copy(row_hbm.at[pl.ds(start, CMAX)], rowv)
    pltpu.sync_copy(col_hbm.at[pl.ds(start, CMAX)], colv)
    nc = jnp.where(cid == 0, C0, C1)
    plsc.subcore_barrier()

    def body(j, carry):
        pltpu.async_copy(q_hbm.at[rowv.at[j]], gbuf, sem).wait()
        pltpu.sync_copy(gbuf, aggs.at[colv.at[j]], add=True)
        return carry

    lax.fori_loop(0, nc, body, 0)
    plsc.subcore_barrier()
    pltpu.sync_copy(aggs.at[sl], out_hbm.at[cid, sl])


@functools.cache
def _sc_aggregate_kernel():
    return pl.kernel(
        _sc_aggregate_body,
        out_type=jax.ShapeDtypeStruct((2, NPAD, 128), jnp.float32),
        mesh=_mesh(),
        scratch_types=[
            pltpu.VMEM((CMAX, CH), jnp.int32),
            pltpu.VMEM((CMAX, CH), jnp.int32),
            pltpu.VMEM((CH, 128), jnp.float32),
            pltpu.VMEM_SHARED((NPAD, 128), jnp.float32),
            pltpu.SemaphoreType.DMA,
        ],
    )


def _sc_aggregate(q, row, col, zeros128):
    return _sc_aggregate_kernel()(q, row, col, zeros128)


_PREC = lax.Precision.HIGHEST


def _leaky(v):
    return jnp.where(v >= 0, v, 0.2 * v)


def _bn_coeffs(s_ref, ss_ref, g_ref, be_ref):
    m = s_ref[...] / N
    var = ss_ref[...] / N - m * m
    a = g_ref[...] * lax.rsqrt(var + 1e-5)
    c = be_ref[...] - m * a
    return a, c


def _row_mask(i, y):
    rowid = lax.broadcasted_iota(jnp.int32, (y.shape[0], 1), 0) + i * RB
    return jnp.where(rowid < N, y, 0.0)


def _accum_stats(i, y, s_ref, ss_ref):
    ym = _row_mask(i, y)
    s = jnp.sum(ym, axis=0)
    ss = jnp.sum(ym * ym, axis=0)

    @pl.when(i == 0)
    def _():
        s_ref[...] = s
        ss_ref[...] = ss

    @pl.when(i > 0)
    def _():
        s_ref[...] = s_ref[...] + s
        ss_ref[...] = ss_ref[...] + ss


def _mm1_body(x_ref, w_ref, b_ref, y_ref, s_ref, ss_ref):
    i = pl.program_id(0)
    y = jnp.dot(x_ref[...], w_ref[...], precision=_PREC,
                preferred_element_type=jnp.float32) + b_ref[...][None, :]
    y_ref[...] = y
    _accum_stats(i, y, s_ref, ss_ref)


def _mm2_body(y1_ref, s1_ref, ss1_ref, g1_ref, be1_ref, w2_ref, b2_ref,
              y2_ref, s2_ref, ss2_ref):
    i = pl.program_id(0)
    a, c = _bn_coeffs(s1_ref, ss1_ref, g1_ref, be1_ref)
    h = _leaky(y1_ref[...] * a[None, :] + c[None, :])
    y2 = jnp.dot(h, w2_ref[...], precision=_PREC,
                 preferred_element_type=jnp.float32) + b2_ref[...][None, :]
    y2_ref[...] = y2
    _accum_stats(i, y2, s2_ref, ss2_ref)


def _dinv_col(degp):
    dd = degp[0] + degp[1] + 1.0
    return lax.rsqrt(dd[:, :1])


def _mm3_body(y2_ref, s2_ref, ss2_ref, g2_ref, be2_ref, wc1_ref, degp_ref,
              q1_ref):
    a, c = _bn_coeffs(s2_ref, ss2_ref, g2_ref, be2_ref)
    h = _leaky(y2_ref[...] * a[None, :] + c[None, :])
    dinv = _dinv_col(degp_ref[...])
    q1_ref[...] = jnp.dot(h, wc1_ref[...], precision=_PREC,
                          preferred_element_type=jnp.float32) * dinv


def _relu_q2_body(sc1_ref, q1_ref, degp_ref, bc1_ref, q2_ref):
    dinv = _dinv_col(degp_ref[...])
    s = sc1_ref[0] + sc1_ref[1] + q1_ref[...]
    h1 = jnp.maximum(s * dinv + bc1_ref[...][None, :], 0.0)
    q2_ref[...] = h1 * dinv


def _final_body(sc2_ref, q2_ref, degp_ref, wmu_ref, bmu_ref, wls_ref,
                bls_ref, eps_ref, mu_ref, ls_ref, z_ref):
    dinv = _dinv_col(degp_ref[...])
    g = (sc2_ref[0] + sc2_ref[1] + q2_ref[...]) * dinv
    mu = jnp.dot(g, wmu_ref[...], precision=_PREC,
                 preferred_element_type=jnp.float32) + bmu_ref[...][None, :]
    ls = jnp.dot(g, wls_ref[...], precision=_PREC,
                 preferred_element_type=jnp.float32) + bls_ref[...][None, :]
    mu_ref[...] = mu
    ls_ref[...] = ls
    z_ref[...] = eps_ref[...] * jnp.exp(ls) + mu


def _full(shape):
    return pl.BlockSpec(shape, lambda i: tuple(0 for _ in shape))


def _rows(width):
    return pl.BlockSpec((RB, width), lambda i: (i, 0))


def _parts(width):
    return pl.BlockSpec((2, RB, width), lambda i: (0, i, 0))


def kernel(x, edge_index, W1, b1, g1, be1, W2, b2, g2, be2, Wc1, bc1,
           Wmu, bmu, Wls, bls, eps):
    f32 = jnp.float32
    x_pad = jnp.pad(x, ((0, NPAD - N), (0, 0)))
    eps_pad = jnp.pad(eps, ((0, NPAD - N), (0, 0)))
    row = jnp.concatenate(
        [edge_index[0], jnp.zeros((EPAD - E,), jnp.int32)]).reshape(NW * NCHUNK, CH)
    col = jnp.concatenate(
        [edge_index[1], jnp.full((EPAD - E,), N, jnp.int32)]).reshape(NW * NCHUNK, CH)
    zeros16 = jnp.zeros((NPAD, 16), f32)
    zeros128 = jnp.zeros((NPAD, 128), f32)
    ones16 = jnp.ones((CH, 16), f32)

    degp = _sc_degree(col, ones16, zeros16)

    y1, s1, ss1 = pl.pallas_call(
        _mm1_body,
        grid=(GRID,),
        in_specs=[_rows(128), _full((128, 1024)), _full((1024,))],
        out_specs=[_rows(1024), _full((1024,)), _full((1024,))],
        out_shape=[jax.ShapeDtypeStruct((NPAD, 1024), f32),
                   jax.ShapeDtypeStruct((1024,), f32),
                   jax.ShapeDtypeStruct((1024,), f32)],
    )(x_pad, W1, b1)

    y2, s2, ss2 = pl.pallas_call(
        _mm2_body,
        grid=(GRID,),
        in_specs=[_rows(1024), _full((1024,)), _full((1024,)),
                  _full((1024,)), _full((1024,)), _full((1024, 512)),
                  _full((512,))],
        out_specs=[_rows(512), _full((512,)), _full((512,))],
        out_shape=[jax.ShapeDtypeStruct((NPAD, 512), f32),
                   jax.ShapeDtypeStruct((512,), f32),
                   jax.ShapeDtypeStruct((512,), f32)],
    )(y1, s1, ss1, g1, be1, W2, b2)

    q1 = pl.pallas_call(
        _mm3_body,
        grid=(GRID,),
        in_specs=[_rows(512), _full((512,)), _full((512,)), _full((512,)),
                  _full((512,)), _full((512, 128)), _parts(16)],
        out_specs=_rows(128),
        out_shape=jax.ShapeDtypeStruct((NPAD, 128), f32),
    )(y2, s2, ss2, g2, be2, Wc1, degp)

    sc1 = _sc_aggregate(q1, row, col, zeros128)

    q2 = pl.pallas_call(
        _relu_q2_body,
        grid=(GRID,),
        in_specs=[_parts(128), _rows(128), _parts(16), _full((128,))],
        out_specs=_rows(128),
        out_shape=jax.ShapeDtypeStruct((NPAD, 128), f32),
    )(sc1, q1, degp, bc1)

    sc2 = _sc_aggregate(q2, row, col, zeros128)

    mu, ls, z = pl.pallas_call(
        _final_body,
        grid=(GRID,),
        in_specs=[_parts(128), _rows(128), _parts(16), _full((128, 64)),
                  _full((64,)), _full((128, 64)), _full((64,)), _rows(64)],
        out_specs=[_rows(64), _rows(64), _rows(64)],
        out_shape=[jax.ShapeDtypeStruct((NPAD, 64), f32),
                   jax.ShapeDtypeStruct((NPAD, 64), f32),
                   jax.ShapeDtypeStruct((NPAD, 64), f32)],
    )(sc2, q2, degp, Wmu, bmu, Wls, bls, eps_pad)

    return (mu[:N], ls[:N], z[:N])


# speedup vs baseline: 1.2665x; 1.2665x over previous
import functools

import jax
import jax.numpy as jnp
from jax import lax
from jax.experimental import pallas as pl
from jax.experimental.pallas import tpu as pltpu
from jax.experimental.pallas import tpu_sc as plsc

N = 10000
NPAD = 10240
E = 160000
NW = 32
CH = 128
NCHUNK = 40
EPAD = NW * NCHUNK * CH
RPT = NPAD // 16
RB = 1024
GRID = NPAD // RB


@functools.cache
def _mesh():
    return plsc.VectorSubcoreMesh(
        core_axis_name="c", subcore_axis_name="s", num_cores=2, num_subcores=16)


def _sc_degree_body(col_hbm, ones_hbm, zeros_hbm, out_hbm, colv, onev, degs):
    cid = lax.axis_index("c")
    sid = lax.axis_index("s")
    wid = cid * 16 + sid
    sl = pl.ds(sid * RPT, RPT)
    pltpu.sync_copy(zeros_hbm.at[sl], degs.at[sl])
    pltpu.sync_copy(col_hbm.at[pl.ds(wid * NCHUNK, NCHUNK)], colv)
    pltpu.sync_copy(ones_hbm, onev)
    plsc.subcore_barrier()

    def body(j, carry):
        pltpu.sync_copy(onev, degs.at[colv.at[j]], add=True)
        return carry

    lax.fori_loop(0, NCHUNK, body, 0)
    plsc.subcore_barrier()
    pltpu.sync_copy(degs.at[sl], out_hbm.at[cid, sl])


@functools.cache
def _sc_degree_kernel():
    return pl.kernel(
        _sc_degree_body,
        out_type=jax.ShapeDtypeStruct((2, NPAD, 16), jnp.float32),
        mesh=_mesh(),
        scratch_types=[
            pltpu.VMEM((NCHUNK, CH), jnp.int32),
            pltpu.VMEM((CH, 16), jnp.float32),
            pltpu.VMEM_SHARED((NPAD, 16), jnp.float32),
        ],
    )


def _sc_degree(col, ones16, zeros16):
    return _sc_degree_kernel()(col, ones16, zeros16)


C0 = 64
C1 = 16
CMAX = max(C0, C1)


def _sc_aggregate_body(q_hbm, row_hbm, col_hbm, zeros_hbm, out_hbm,
                       rowv, colv, gbuf, aggs, sem):
    cid = lax.axis_index("c")
    sid = lax.axis_index("s")
    sl = pl.ds(sid * RPT, RPT)
    pltpu.sync_copy(zeros_hbm.at[sl], aggs.at[sl])

    start = jnp.where(cid == 0, sid * C0, 16 * C0 + sid * C1)
    start = pl.multiple_of(start, 8)
    pltpu.sync_copy(row_hbm.at[pl.ds(start, CMAX)], rowv)
    pltpu.sync_copy(col_hbm.at[pl.ds(start, CMAX)], colv)
    nc = jnp.where(cid == 0, C0, C1)
    plsc.subcore_barrier()

    def body(j, carry):
        pltpu.async_copy(q_hbm.at[rowv.at[j]], gbuf, sem).wait()
        pltpu.sync_copy(gbuf, aggs.at[colv.at[j]], add=True)
        return carry

    lax.fori_loop(0, nc, body, 0)
    plsc.subcore_barrier()
    pltpu.sync_copy(aggs.at[sl], out_hbm.at[cid, sl])


@functools.cache
def _sc_aggregate_kernel():
    return pl.kernel(
        _sc_aggregate_body,
        out_type=jax.ShapeDtypeStruct((2, NPAD, 128), jnp.float32),
        mesh=_mesh(),
        scratch_types=[
            pltpu.VMEM((CMAX, CH), jnp.int32),
            pltpu.VMEM((CMAX, CH), jnp.int32),
            pltpu.VMEM((CH, 128), jnp.float32),
            pltpu.VMEM_SHARED((NPAD, 128), jnp.float32),
            pltpu.SemaphoreType.DMA,
        ],
    )


def _sc_aggregate(q, row, col, zeros128):
    return _sc_aggregate_kernel()(q, row, col, zeros128)


_PREC = lax.Precision.HIGHEST


def _leaky(v):
    return jnp.where(v >= 0, v, 0.2 * v)


def _bn_coeffs(s_ref, ss_ref, g_ref, be_ref):
    m = s_ref[...] / N
    var = ss_ref[...] / N - m * m
    a = g_ref[...] * lax.rsqrt(var + 1e-5)
    c = be_ref[...] - m * a
    return a, c


def _row_mask(i, y):
    rowid = lax.broadcasted_iota(jnp.int32, (y.shape[0], 1), 0) + i * RB
    return jnp.where(rowid < N, y, 0.0)


def _accum_stats(i, y, s_ref, ss_ref):
    ym = _row_mask(i, y)
    s = jnp.sum(ym, axis=0)
    ss = jnp.sum(ym * ym, axis=0)

    @pl.when(i == 0)
    def _():
        s_ref[...] = s
        ss_ref[...] = ss

    @pl.when(i > 0)
    def _():
        s_ref[...] = s_ref[...] + s
        ss_ref[...] = ss_ref[...] + ss


def _mm1_body(x_ref, w_ref, b_ref, y_ref, s_ref, ss_ref):
    i = pl.program_id(0)
    y = jnp.dot(x_ref[...], w_ref[...], precision=_PREC,
                preferred_element_type=jnp.float32) + b_ref[...][None, :]
    y_ref[...] = y
    _accum_stats(i, y, s_ref, ss_ref)


def _mm2_body(y1_ref, s1_ref, ss1_ref, g1_ref, be1_ref, w2_ref, b2_ref,
              y2_ref, s2_ref, ss2_ref):
    i = pl.program_id(0)
    a, c = _bn_coeffs(s1_ref, ss1_ref, g1_ref, be1_ref)
    h = _leaky(y1_ref[...] * a[None, :] + c[None, :])
    y2 = jnp.dot(h, w2_ref[...], precision=_PREC,
                 preferred_element_type=jnp.float32) + b2_ref[...][None, :]
    y2_ref[...] = y2
    _accum_stats(i, y2, s2_ref, ss2_ref)


def _dinv_col(degp):
    dd = degp[0] + degp[1] + 1.0
    return lax.rsqrt(dd[:, :1])


def _mm3_body(y2_ref, s2_ref, ss2_ref, g2_ref, be2_ref, wc1_ref, degp_ref,
              q1_ref):
    a, c = _bn_coeffs(s2_ref, ss2_ref, g2_ref, be2_ref)
    h = _leaky(y2_ref[...] * a[None, :] + c[None, :])
    dinv = _dinv_col(degp_ref[...])
    q1_ref[...] = jnp.dot(h, wc1_ref[...], precision=_PREC,
                          preferred_element_type=jnp.float32) * dinv


def _relu_q2_body(sc1_ref, q1_ref, degp_ref, bc1_ref, q2_ref):
    dinv = _dinv_col(degp_ref[...])
    s = sc1_ref[0] + sc1_ref[1] + q1_ref[...]
    h1 = jnp.maximum(s * dinv + bc1_ref[...][None, :], 0.0)
    q2_ref[...] = h1 * dinv


def _final_body(sc2_ref, q2_ref, degp_ref, wmu_ref, bmu_ref, wls_ref,
                bls_ref, eps_ref, mu_ref, ls_ref, z_ref):
    dinv = _dinv_col(degp_ref[...])
    g = (sc2_ref[0] + sc2_ref[1] + q2_ref[...]) * dinv
    mu = jnp.dot(g, wmu_ref[...], precision=_PREC,
                 preferred_element_type=jnp.float32) + bmu_ref[...][None, :]
    ls = jnp.dot(g, wls_ref[...], precision=_PREC,
                 preferred_element_type=jnp.float32) + bls_ref[...][None, :]
    mu_ref[...] = mu
    ls_ref[...] = ls
    z_ref[...] = eps_ref[...] * jnp.exp(ls) + mu


def _full(shape):
    return pl.BlockSpec(shape, lambda i: tuple(0 for _ in shape))


def _rows(width):
    return pl.BlockSpec((RB, width), lambda i: (i, 0))


def _parts(width):
    return pl.BlockSpec((2, RB, width), lambda i: (0, i, 0))


def kernel(x, edge_index, W1, b1, g1, be1, W2, b2, g2, be2, Wc1, bc1,
           Wmu, bmu, Wls, bls, eps):
    f32 = jnp.float32
    x_pad = jnp.pad(x, ((0, NPAD - N), (0, 0)))
    eps_pad = jnp.pad(eps, ((0, NPAD - N), (0, 0)))
    row = jnp.concatenate(
        [edge_index[0], jnp.zeros((EPAD - E,), jnp.int32)]).reshape(NW * NCHUNK, CH)
    col = jnp.concatenate(
        [edge_index[1], jnp.full((EPAD - E,), N, jnp.int32)]).reshape(NW * NCHUNK, CH)
    zeros16 = jnp.zeros((NPAD, 16), f32)
    zeros128 = jnp.zeros((NPAD, 128), f32)
    ones16 = jnp.ones((CH, 16), f32)

    degp = _sc_degree(col, ones16, zeros16)

    y1, s1, ss1 = pl.pallas_call(
        _mm1_body,
        grid=(GRID,),
        in_specs=[_rows(128), _full((128, 1024)), _full((1024,))],
        out_specs=[_rows(1024), _full((1024,)), _full((1024,))],
        out_shape=[jax.ShapeDtypeStruct((NPAD, 1024), f32),
                   jax.ShapeDtypeStruct((1024,), f32),
                   jax.ShapeDtypeStruct((1024,), f32)],
    )(x_pad, W1, b1)

    y2, s2, ss2 = pl.pallas_call(
        _mm2_body,
        grid=(GRID,),
        in_specs=[_rows(1024), _full((1024,)), _full((1024,)),
                  _full((1024,)), _full((1024,)), _full((1024, 512)),
                  _full((512,))],
        out_specs=[_rows(512), _full((512,)), _full((512,))],
        out_shape=[jax.ShapeDtypeStruct((NPAD, 512), f32),
                   jax.ShapeDtypeStruct((512,), f32),
                   jax.ShapeDtypeStruct((512,), f32)],
    )(y1, s1, ss1, g1, be1, W2, b2)

    q1 = pl.pallas_call(
        _mm3_body,
        grid=(GRID,),
        in_specs=[_rows(512), _full((512,)), _full((512,)), _full((512,)),
                  _full((512,)), _full((512, 128)), _parts(16)],
        out_specs=_rows(128),
        out_shape=jax.ShapeDtypeStruct((NPAD, 128), f32),
    )(y2, s2, ss2, g2, be2, Wc1, degp)

    sc1 = _sc_aggregate(q1, row, col, zeros128)

    q2 = pl.pallas_call(
        _relu_q2_body,
        grid=(GRID,),
        in_specs=[_parts(128), _rows(128), _parts(16), _full((128,))],
        out_specs=_rows(128),
        out_shape=jax.ShapeDtypeStruct((NPAD, 128), f32),
    )(sc1, q1, degp, bc1)

    sc2 = _sc_aggregate(q2, row, col, zeros128)

    mu, ls, z = pl.pallas_call(
        _final_body,
        grid=(GRID,),
        in_specs=[_parts(128), _rows(128), _parts(16), _full((128, 64)),
                  _full((64,)), _full((128, 64)), _full((64,)), _rows(64)],
        out_specs=[_rows(64), _rows(64), _rows(64)],
        out_shape=[jax.ShapeDtypeStruct((NPAD, 64), f32),
                   jax.ShapeDtypeStruct((NPAD, 64), f32),
                   jax.ShapeDtypeStruct((NPAD, 64), f32)],
    )(sc2, q2, degp, Wmu, bmu, Wls, bls, eps_pad)

    return (mu[:N], ls[:N], z[:N])
